# Initial kernel scaffold; baseline (speedup 1.0000x reference)
import jax
import jax.numpy as jnp
from jax.experimental import pallas as pl


def kernel(x, Wr, br, W1, b1, W2, b2):
    raise NotImplementedError("write your pallas kernel here")


# speedup vs baseline: 1.0619x; 1.0619x over previous
import functools

import jax
import jax.numpy as jnp
from jax import lax
from jax.experimental import pallas as pl
from jax.experimental.pallas import tpu as pltpu
import jax.experimental.pallas.tpu_sc as plsc

H = 1024
F = 2048
E = 8
N = 4096
K = 2
PAIRS = N * K
BLK = 256
CAP = PAIRS + E * BLK
NBLK = CAP // BLK
NW = 32
SLOTS_W = CAP // NW
GCH = 64
TOK_W = N // NW
CCH = 32


def _router_body(x_ref, wr_ref, br_ref, pos_ref, gates_ref, meta_ref):
    xv = x_ref[...]
    wr = wr_ref[...]
    logits = lax.dot_general(xv, wr, (((1,), (1,)), ((), ())),
                             preferred_element_type=jnp.float32)
    logits = logits + br_ref[...]
    iota8 = lax.broadcasted_iota(jnp.int32, (N, E), 1)
    m1 = jnp.max(logits, axis=1, keepdims=True)
    i1 = jnp.min(jnp.where(logits == m1, iota8, E), axis=1)
    masked = jnp.where(iota8 == i1[:, None], jnp.float32(-1e30), logits)
    m2 = jnp.max(masked, axis=1, keepdims=True)
    i2 = jnp.min(jnp.where(masked == m2, iota8, E), axis=1)
    g1 = 1.0 / (1.0 + jnp.exp(m2[:, 0] - m1[:, 0]))
    g2 = 1.0 - g1
    gates_ref[...] = jnp.concatenate(
        [g1.reshape(N // 128, 128), g2.reshape(N // 128, 128)], axis=0)

    pair_e = jnp.concatenate(
        [i1.reshape(N // 128, 128), i2.reshape(N // 128, 128)], axis=0)
    PR = PAIRS // 128

    offs, ends = [], []
    run = jnp.int32(0)
    for e in range(E):
        ce = jnp.sum((pair_e == e).astype(jnp.float32)).astype(jnp.int32)
        offs.append(run)
        run = run + ((ce + BLK - 1) // BLK) * BLK
        ends.append(run)

    ri = lax.broadcasted_iota(jnp.int32, (128, 128), 0)
    ci = lax.broadcasted_iota(jnp.int32, (128, 128), 1)
    U = (ri <= ci).astype(jnp.float32)
    rA = lax.broadcasted_iota(jnp.int32, (PR, PR), 0)
    cA = lax.broadcasted_iota(jnp.int32, (PR, PR), 1)
    A = (rA > cA).astype(jnp.float32)
    pos_acc = jnp.zeros((PR, 128), jnp.float32)
    for e in range(E):
        Me = (pair_e == e).astype(jnp.float32)
        R = lax.dot_general(Me, U, (((1,), (0,)), ((), ())),
                            preferred_element_type=jnp.float32)
        prev = lax.dot_general(A, R, (((1,), (0,)), ((), ())),
                               preferred_element_type=jnp.float32)
        rank = R + prev[:, 127:128] - Me
        pos_acc = pos_acc + Me * (offs[e].astype(jnp.float32) + rank)
    pos_ref[...] = pos_acc.astype(jnp.int32)

    bstart = lax.broadcasted_iota(jnp.int32, (1, 128), 1) * BLK
    be = jnp.zeros((1, 128), jnp.int32)
    for e in range(E):
        be = be + (bstart >= ends[e]).astype(jnp.int32)
    be = jnp.minimum(be, E - 1)
    act = (bstart < run).astype(jnp.int32)
    meta_ref[...] = jnp.concatenate(
        [be, act] + [jnp.zeros((1, 128), jnp.int32)] * 6, axis=0)


def _router_call(x2d, Wr, br2):
    return pl.pallas_call(
        _router_body,
        in_specs=[
            pl.BlockSpec((N, H), lambda: (0, 0)),
            pl.BlockSpec((E, H), lambda: (0, 0)),
            pl.BlockSpec((1, E), lambda: (0, 0)),
        ],
        out_specs=[
            pl.BlockSpec((PAIRS // 128, 128), lambda: (0, 0)),
            pl.BlockSpec((PAIRS // 128, 128), lambda: (0, 0)),
            pl.BlockSpec((8, 128), lambda: (0, 0)),
        ],
        out_shape=[
            jax.ShapeDtypeStruct((PAIRS // 128, 128), jnp.int32),
            jax.ShapeDtypeStruct((PAIRS // 128, 128), jnp.float32),
            jax.ShapeDtypeStruct((8, 128), jnp.int32),
        ],
    )(x2d, Wr, br2)


@functools.cache
def _sc_mesh():
    return plsc.VectorSubcoreMesh(core_axis_name="c", subcore_axis_name="s")


@functools.cache
def _dispatch_kernel():
    return pl.kernel(
        _dispatch_body,
        mesh=_sc_mesh(),
        out_type=(jax.ShapeDtypeStruct((CAP, H), jnp.float32),
                  jax.ShapeDtypeStruct((CAP,), jnp.float32)),
        scratch_types=[
            pltpu.VMEM((PAIRS,), jnp.int32),
            pltpu.VMEM((PAIRS,), jnp.float32),
            pltpu.VMEM((SLOTS_W,), jnp.int32),
            pltpu.VMEM((SLOTS_W,), jnp.float32),
            pltpu.VMEM((GCH, H), jnp.float32),
            pltpu.SemaphoreType.DMA,
        ],
        compiler_params=pltpu.CompilerParams(needs_layout_passes=False),
    )


def _dispatch_body(pos_hbm, gat_hbm, x_hbm, xg_hbm, gsl_hbm,
                   posv, gatv, ids, gl, rowbuf, sem):
    wid = lax.axis_index("s") * 2 + lax.axis_index("c")
    base = wid * SLOTS_W
    pltpu.sync_copy(pos_hbm, posv)
    pltpu.sync_copy(gat_hbm, gatv)

    def zbody(j, carry):
        ids[pl.ds(j * 16, 16)] = jnp.zeros((16,), jnp.int32)
        gl[pl.ds(j * 16, 16)] = jnp.zeros((16,), jnp.float32)
        return carry
    lax.fori_loop(0, SLOTS_W // 16, zbody, 0)

    def sbody(j, carry):
        pv = posv[pl.ds(j * 16, 16)]
        gv = gatv[pl.ds(j * 16, 16)]
        pidx = lax.iota(jnp.int32, 16) + j * 16
        tok = jnp.bitwise_and(pidx, N - 1)
        loc = pv - base
        msk = jnp.logical_and(pv >= base, pv < base + SLOTS_W)
        plsc.store_scatter(ids, [loc], tok, mask=msk)
        plsc.store_scatter(gl, [loc], gv, mask=msk)
        return carry
    lax.fori_loop(0, PAIRS // 16, sbody, 0)

    pltpu.sync_copy(gl, gsl_hbm.at[pl.ds(base, SLOTS_W)])
    for c in range(SLOTS_W // GCH):
        pltpu.async_copy(x_hbm.at[ids.at[pl.ds(c * GCH, GCH)]],
                         rowbuf, sem).wait()
        pltpu.sync_copy(rowbuf, xg_hbm.at[pl.ds(base + c * GCH, GCH)])


def _ffn_body(m_ref, xg_ref, w1_ref, b1_ref, w2_ref, b2_ref, g_ref, y_ref):
    i = pl.program_id(0)

    @pl.when(m_ref[1, i] == 1)
    def _():
        xb = xg_ref[...]
        h = lax.dot_general(xb, w1_ref[0], (((1,), (1,)), ((), ())),
                            preferred_element_type=jnp.float32)
        h = jnp.maximum(h + b1_ref[0], 0.0)
        y = lax.dot_general(h, w2_ref[0], (((1,), (1,)), ((), ())),
                            preferred_element_type=jnp.float32)
        y = y + b2_ref[0]
        g = g_ref[0, 0, :]
        y_ref[...] = y * g[:, None]


def _ffn_call(meta, xg, W1, b1r, W2, b2r, g3d):
    grid_spec = pltpu.PrefetchScalarGridSpec(
        num_scalar_prefetch=1,
        grid=(NBLK,),
        in_specs=[
            pl.BlockSpec((BLK, H), lambda i, m: (i, 0)),
            pl.BlockSpec((1, F, H), lambda i, m: (m[0, i], 0, 0)),
            pl.BlockSpec((1, 1, F), lambda i, m: (m[0, i], 0, 0)),
            pl.BlockSpec((1, H, F), lambda i, m: (m[0, i], 0, 0)),
            pl.BlockSpec((1, 1, H), lambda i, m: (m[0, i], 0, 0)),
            pl.BlockSpec((1, 1, BLK), lambda i, m: (i, 0, 0)),
        ],
        out_specs=pl.BlockSpec((BLK, H), lambda i, m: (i, 0)),
    )
    return pl.pallas_call(
        _ffn_body,
        grid_spec=grid_spec,
        out_shape=jax.ShapeDtypeStruct((CAP, H), jnp.float32),
        compiler_params=pltpu.CompilerParams(
            dimension_semantics=("arbitrary",)),
    )(meta, xg, W1, b1r, W2, b2r, g3d)


@functools.cache
def _combine_kernel():
    return pl.kernel(
        _combine_body,
        mesh=_sc_mesh(),
        out_type=jax.ShapeDtypeStruct((N, H), jnp.float32),
        scratch_types=[
            pltpu.VMEM((TOK_W,), jnp.int32),
            pltpu.VMEM((TOK_W,), jnp.int32),
            pltpu.VMEM((CCH, H), jnp.float32),
            pltpu.VMEM((CCH, H), jnp.float32),
            pltpu.SemaphoreType.DMA,
            pltpu.SemaphoreType.DMA,
        ],
        compiler_params=pltpu.CompilerParams(needs_layout_passes=False),
    )


def _combine_body(p0_hbm, p1_hbm, y_hbm, out_hbm, p0v, p1v, bufa, bufb, sa, sb):
    wid = lax.axis_index("s") * 2 + lax.axis_index("c")
    base = wid * TOK_W
    pltpu.sync_copy(p0_hbm.at[pl.ds(base, TOK_W)], p0v)
    pltpu.sync_copy(p1_hbm.at[pl.ds(base, TOK_W)], p1v)
    for c in range(TOK_W // CCH):
        da = pltpu.async_copy(y_hbm.at[p0v.at[pl.ds(c * CCH, CCH)]], bufa, sa)
        db = pltpu.async_copy(y_hbm.at[p1v.at[pl.ds(c * CCH, CCH)]], bufb, sb)
        da.wait()
        db.wait()

        def abody(j, carry):
            r = j // (H // 16)
            k = j % (H // 16)
            av = bufa[r, pl.ds(k * 16, 16)]
            bv = bufb[r, pl.ds(k * 16, 16)]
            bufa[r, pl.ds(k * 16, 16)] = av + bv
            return carry
        lax.fori_loop(0, CCH * (H // 16), abody, 0)
        pltpu.sync_copy(bufa, out_hbm.at[pl.ds(base + c * CCH, CCH)])


def kernel(x, Wr, br, W1, b1, W2, b2):
    x2d = x.reshape(N, H)
    pos_m, gates_m, meta = _router_call(x2d, Wr, br.reshape(1, E))
    pos_pairs = pos_m.reshape(-1)
    gates_pairs = gates_m.reshape(-1)
    xg, gsl = _dispatch_kernel()(pos_pairs, gates_pairs, x2d)
    y = _ffn_call(meta, xg, W1, b1.reshape(E, 1, F), W2, b2.reshape(E, 1, H),
                  gsl.reshape(NBLK, 1, BLK))
    out2d = _combine_kernel()(pos_pairs[:N], pos_pairs[N:], y)
    return out2d.reshape(x.shape)


# speedup vs baseline: 1.5377x; 1.4480x over previous
import functools

import jax
import jax.numpy as jnp
from jax import lax
from jax.experimental import pallas as pl
from jax.experimental.pallas import tpu as pltpu
import jax.experimental.pallas.tpu_sc as plsc

H = 1024
F = 2048
E = 8
N = 4096
K = 2
PAIRS = N * K
BLK = 256
CAP = PAIRS + E * BLK
NBLK = CAP // BLK
NW = 32
SLOTS_W = CAP // NW
GCH = 32
TOK_W = N // NW
CCH = 32


def _router_body(x_ref, wr_ref, br_ref, pos_ref, gates_ref, meta_ref):
    xv = x_ref[...]
    wr = wr_ref[...]
    logits = lax.dot_general(xv, wr, (((1,), (1,)), ((), ())),
                             preferred_element_type=jnp.float32)
    logits = logits + br_ref[...]
    iota8 = lax.broadcasted_iota(jnp.int32, (N, E), 1)
    m1 = jnp.max(logits, axis=1, keepdims=True)
    i1 = jnp.min(jnp.where(logits == m1, iota8, E), axis=1)
    masked = jnp.where(iota8 == i1[:, None], jnp.float32(-1e30), logits)
    m2 = jnp.max(masked, axis=1, keepdims=True)
    i2 = jnp.min(jnp.where(masked == m2, iota8, E), axis=1)
    g1 = 1.0 / (1.0 + jnp.exp(m2[:, 0] - m1[:, 0]))
    g2 = 1.0 - g1
    gates_ref[...] = jnp.concatenate(
        [g1.reshape(N // 128, 128), g2.reshape(N // 128, 128)], axis=0)

    pair_e = jnp.concatenate(
        [i1.reshape(N // 128, 128), i2.reshape(N // 128, 128)], axis=0)
    PR = PAIRS // 128

    offs, ends = [], []
    run = jnp.int32(0)
    for e in range(E):
        ce = jnp.sum((pair_e == e).astype(jnp.float32)).astype(jnp.int32)
        offs.append(run)
        run = run + ((ce + BLK - 1) // BLK) * BLK
        ends.append(run)

    ri = lax.broadcasted_iota(jnp.int32, (128, 128), 0)
    ci = lax.broadcasted_iota(jnp.int32, (128, 128), 1)
    U = (ri <= ci).astype(jnp.float32)
    rA = lax.broadcasted_iota(jnp.int32, (PR, PR), 0)
    cA = lax.broadcasted_iota(jnp.int32, (PR, PR), 1)
    A = (rA > cA).astype(jnp.float32)
    pos_acc = jnp.zeros((PR, 128), jnp.float32)
    for e in range(E):
        Me = (pair_e == e).astype(jnp.float32)
        R = lax.dot_general(Me, U, (((1,), (0,)), ((), ())),
                            preferred_element_type=jnp.float32)
        prev = lax.dot_general(A, R, (((1,), (0,)), ((), ())),
                               preferred_element_type=jnp.float32)
        rank = R + prev[:, 127:128] - Me
        pos_acc = pos_acc + Me * (offs[e].astype(jnp.float32) + rank)
    pos_ref[...] = pos_acc.astype(jnp.int32)

    bstart = lax.broadcasted_iota(jnp.int32, (1, 128), 1) * BLK
    be = jnp.zeros((1, 128), jnp.int32)
    for e in range(E):
        be = be + (bstart >= ends[e]).astype(jnp.int32)
    be = jnp.minimum(be, E - 1)
    act = (bstart < run).astype(jnp.int32)
    meta_ref[...] = jnp.concatenate(
        [be, act] + [jnp.zeros((1, 128), jnp.int32)] * 6, axis=0)


def _router_call(x2d, Wr, br2):
    return pl.pallas_call(
        _router_body,
        in_specs=[
            pl.BlockSpec((N, H), lambda: (0, 0)),
            pl.BlockSpec((E, H), lambda: (0, 0)),
            pl.BlockSpec((1, E), lambda: (0, 0)),
        ],
        out_specs=[
            pl.BlockSpec((PAIRS // 128, 128), lambda: (0, 0)),
            pl.BlockSpec((PAIRS // 128, 128), lambda: (0, 0)),
            pl.BlockSpec((8, 128), lambda: (0, 0)),
        ],
        out_shape=[
            jax.ShapeDtypeStruct((PAIRS // 128, 128), jnp.int32),
            jax.ShapeDtypeStruct((PAIRS // 128, 128), jnp.float32),
            jax.ShapeDtypeStruct((8, 128), jnp.int32),
        ],
    )(x2d, Wr, br2)


@functools.cache
def _sc_mesh():
    return plsc.VectorSubcoreMesh(core_axis_name="c", subcore_axis_name="s")


PCH = PAIRS // NW // GCH


@functools.cache
def _dispatch_kernel():
    return pl.kernel(
        _dispatch_body,
        mesh=_sc_mesh(),
        out_type=(jax.ShapeDtypeStruct((CAP, H), jnp.float32),
                  jax.ShapeDtypeStruct((CAP,), jnp.float32)),
        scratch_types=[
            pltpu.VMEM((PCH, GCH), jnp.int32),
            pltpu.VMEM((PCH, GCH), jnp.float32),
            pltpu.VMEM((GCH, H), jnp.float32),
            pltpu.VMEM((GCH, H), jnp.float32),
            pltpu.SemaphoreType.DMA,
            pltpu.SemaphoreType.DMA,
            pltpu.SemaphoreType.DMA,
        ],
        compiler_params=pltpu.CompilerParams(needs_layout_passes=False),
    )


def _dispatch_body(pos_hbm, gat_hbm, x_hbm, xg_hbm, gsl_hbm,
                   posv, gatv, buf0, buf1, sem0, sem1, semg):
    wid = lax.axis_index("s") * 2 + lax.axis_index("c")
    t0 = jnp.bitwise_and(wid, 15) * (PCH * GCH)
    pltpu.sync_copy(pos_hbm.at[wid], posv)
    pltpu.sync_copy(gat_hbm.at[wid], gatv)
    bufs = (buf0, buf1)
    sems = (sem0, sem1)
    descs = [None, None]
    gdescs = []
    for c in range(PCH):
        b = bufs[c % 2]
        if descs[c % 2] is not None:
            descs[c % 2].wait()
        pltpu.sync_copy(x_hbm.at[pl.ds(t0 + c * GCH, GCH)], b)
        descs[c % 2] = pltpu.async_copy(b, xg_hbm.at[posv.at[c]], sems[c % 2])
        gdescs.append(pltpu.async_copy(gatv.at[c], gsl_hbm.at[posv.at[c]],
                                       semg))
    for d in descs:
        d.wait()
    for d in gdescs:
        d.wait()


def _ffn_body(m_ref, xg_ref, w1_ref, b1_ref, w2_ref, b2_ref, g_ref, y_ref):
    i = pl.program_id(0)

    @pl.when(m_ref[1, i] == 1)
    def _():
        xb = xg_ref[...]
        h = lax.dot_general(xb, w1_ref[0], (((1,), (1,)), ((), ())),
                            preferred_element_type=jnp.float32)
        h = jnp.maximum(h + b1_ref[0], 0.0)
        y = lax.dot_general(h, w2_ref[0], (((1,), (1,)), ((), ())),
                            preferred_element_type=jnp.float32)
        y = y + b2_ref[0]
        g = g_ref[0, 0, :]
        y_ref[...] = y * g[:, None]


def _ffn_call(meta, xg, W1, b1r, W2, b2r, g3d):
    grid_spec = pltpu.PrefetchScalarGridSpec(
        num_scalar_prefetch=1,
        grid=(NBLK,),
        in_specs=[
            pl.BlockSpec((BLK, H), lambda i, m: (i, 0)),
            pl.BlockSpec((1, F, H), lambda i, m: (m[0, i], 0, 0)),
            pl.BlockSpec((1, 1, F), lambda i, m: (m[0, i], 0, 0)),
            pl.BlockSpec((1, H, F), lambda i, m: (m[0, i], 0, 0)),
            pl.BlockSpec((1, 1, H), lambda i, m: (m[0, i], 0, 0)),
            pl.BlockSpec((1, 1, BLK), lambda i, m: (i, 0, 0)),
        ],
        out_specs=pl.BlockSpec((BLK, H), lambda i, m: (i, 0)),
    )
    return pl.pallas_call(
        _ffn_body,
        grid_spec=grid_spec,
        out_shape=jax.ShapeDtypeStruct((CAP, H), jnp.float32),
        compiler_params=pltpu.CompilerParams(
            dimension_semantics=("arbitrary",)),
    )(meta, xg, W1, b1r, W2, b2r, g3d)


@functools.cache
def _combine_kernel():
    return pl.kernel(
        _combine_body,
        mesh=_sc_mesh(),
        out_type=jax.ShapeDtypeStruct((N, H), jnp.float32),
        scratch_types=[
            pltpu.VMEM((TOK_W,), jnp.int32),
            pltpu.VMEM((TOK_W,), jnp.int32),
            pltpu.VMEM((CCH, H), jnp.float32),
            pltpu.VMEM((CCH, H), jnp.float32),
            pltpu.SemaphoreType.DMA,
            pltpu.SemaphoreType.DMA,
        ],
        compiler_params=pltpu.CompilerParams(needs_layout_passes=False),
    )


def _combine_body(p0_hbm, p1_hbm, y_hbm, out_hbm, p0v, p1v, bufa, bufb, sa, sb):
    wid = lax.axis_index("s") * 2 + lax.axis_index("c")
    base = wid * TOK_W
    pltpu.sync_copy(p0_hbm.at[pl.ds(base, TOK_W)], p0v)
    pltpu.sync_copy(p1_hbm.at[pl.ds(base, TOK_W)], p1v)
    for c in range(TOK_W // CCH):
        da = pltpu.async_copy(y_hbm.at[p0v.at[pl.ds(c * CCH, CCH)]], bufa, sa)
        db = pltpu.async_copy(y_hbm.at[p1v.at[pl.ds(c * CCH, CCH)]], bufb, sb)
        da.wait()
        db.wait()

        @plsc.parallel_loop(0, CCH * (H // 16), unroll=8)
        def _add(j):
            r = j // (H // 16)
            k = j % (H // 16)
            av = bufa[r, pl.ds(k * 16, 16)]
            bv = bufb[r, pl.ds(k * 16, 16)]
            bufa[r, pl.ds(k * 16, 16)] = av + bv
        pltpu.sync_copy(bufa, out_hbm.at[pl.ds(base + c * CCH, CCH)])


def kernel(x, Wr, br, W1, b1, W2, b2):
    x2d = x.reshape(N, H)
    pos_m, gates_m, meta = _router_call(x2d, Wr, br.reshape(1, E))
    pos_pairs = pos_m.reshape(-1)
    gates_pairs = gates_m.reshape(-1)
    xg, gsl = _dispatch_kernel()(pos_pairs.reshape(NW, PCH, GCH),
                                 gates_pairs.reshape(NW, PCH, GCH), x2d)
    y = _ffn_call(meta, xg, W1, b1.reshape(E, 1, F), W2, b2.reshape(E, 1, H),
                  gsl.reshape(NBLK, 1, BLK))
    out2d = _combine_kernel()(pos_pairs[:N], pos_pairs[N:], y)
    return out2d.reshape(x.shape)


# speedup vs baseline: 1.5881x; 1.0328x over previous
import functools

import jax
import jax.numpy as jnp
from jax import lax
from jax.experimental import pallas as pl
from jax.experimental.pallas import tpu as pltpu
import jax.experimental.pallas.tpu_sc as plsc

H = 1024
F = 2048
E = 8
N = 4096
K = 2
PAIRS = N * K
BLK = 256
CAP = PAIRS + E * BLK
NBLK = CAP // BLK
NW = 32
SLOTS_W = CAP // NW
GCH = 32
TOK_W = N // NW
CCH = 16


def _router_body(x_ref, wr_ref, br_ref, pos_ref, gates_ref, meta_ref):
    xv = x_ref[...]
    wr = wr_ref[...]
    logits = lax.dot_general(xv, wr, (((1,), (1,)), ((), ())),
                             preferred_element_type=jnp.float32)
    logits = logits + br_ref[...]
    iota8 = lax.broadcasted_iota(jnp.int32, (N, E), 1)
    m1 = jnp.max(logits, axis=1, keepdims=True)
    i1 = jnp.min(jnp.where(logits == m1, iota8, E), axis=1)
    masked = jnp.where(iota8 == i1[:, None], jnp.float32(-1e30), logits)
    m2 = jnp.max(masked, axis=1, keepdims=True)
    i2 = jnp.min(jnp.where(masked == m2, iota8, E), axis=1)
    g1 = 1.0 / (1.0 + jnp.exp(m2[:, 0] - m1[:, 0]))
    g2 = 1.0 - g1
    gates_ref[...] = jnp.concatenate(
        [g1.reshape(N // 128, 128), g2.reshape(N // 128, 128)], axis=0)

    pair_e = jnp.concatenate(
        [i1.reshape(N // 128, 128), i2.reshape(N // 128, 128)], axis=0)
    PR = PAIRS // 128

    offs, ends = [], []
    run = jnp.int32(0)
    for e in range(E):
        ce = jnp.sum((pair_e == e).astype(jnp.float32)).astype(jnp.int32)
        offs.append(run)
        run = run + ((ce + BLK - 1) // BLK) * BLK
        ends.append(run)

    ri = lax.broadcasted_iota(jnp.int32, (128, 128), 0)
    ci = lax.broadcasted_iota(jnp.int32, (128, 128), 1)
    U = (ri <= ci).astype(jnp.float32)
    rA = lax.broadcasted_iota(jnp.int32, (PR, PR), 0)
    cA = lax.broadcasted_iota(jnp.int32, (PR, PR), 1)
    A = (rA > cA).astype(jnp.float32)
    pos_acc = jnp.zeros((PR, 128), jnp.float32)
    for e in range(E):
        Me = (pair_e == e).astype(jnp.float32)
        R = lax.dot_general(Me, U, (((1,), (0,)), ((), ())),
                            preferred_element_type=jnp.float32)
        prev = lax.dot_general(A, R, (((1,), (0,)), ((), ())),
                               preferred_element_type=jnp.float32)
        rank = R + prev[:, 127:128] - Me
        pos_acc = pos_acc + Me * (offs[e].astype(jnp.float32) + rank)
    pos_ref[...] = pos_acc.astype(jnp.int32)

    bstart = lax.broadcasted_iota(jnp.int32, (1, 128), 1) * BLK
    be = jnp.zeros((1, 128), jnp.int32)
    for e in range(E):
        be = be + (bstart >= ends[e]).astype(jnp.int32)
    be = jnp.minimum(be, E - 1)
    act = (bstart < run).astype(jnp.int32)
    meta_ref[...] = jnp.concatenate(
        [be, act] + [jnp.zeros((1, 128), jnp.int32)] * 6, axis=0)


def _router_call(x2d, Wr, br2):
    return pl.pallas_call(
        _router_body,
        in_specs=[
            pl.BlockSpec((N, H), lambda: (0, 0)),
            pl.BlockSpec((E, H), lambda: (0, 0)),
            pl.BlockSpec((1, E), lambda: (0, 0)),
        ],
        out_specs=[
            pl.BlockSpec((PAIRS // 128, 128), lambda: (0, 0)),
            pl.BlockSpec((PAIRS // 128, 128), lambda: (0, 0)),
            pl.BlockSpec((8, 128), lambda: (0, 0)),
        ],
        out_shape=[
            jax.ShapeDtypeStruct((PAIRS // 128, 128), jnp.int32),
            jax.ShapeDtypeStruct((PAIRS // 128, 128), jnp.float32),
            jax.ShapeDtypeStruct((8, 128), jnp.int32),
        ],
    )(x2d, Wr, br2)


@functools.cache
def _sc_mesh():
    return plsc.VectorSubcoreMesh(core_axis_name="c", subcore_axis_name="s")


PCH = PAIRS // NW // GCH


@functools.cache
def _dispatch_kernel():
    return pl.kernel(
        _dispatch_body,
        mesh=_sc_mesh(),
        out_type=(jax.ShapeDtypeStruct((CAP, H), jnp.float32),
                  jax.ShapeDtypeStruct((CAP,), jnp.float32)),
        scratch_types=[
            pltpu.VMEM((PCH, GCH), jnp.int32),
            pltpu.VMEM((PCH, GCH), jnp.float32),
            pltpu.VMEM((GCH, H), jnp.float32),
            pltpu.VMEM((GCH, H), jnp.float32),
            pltpu.SemaphoreType.DMA,
            pltpu.SemaphoreType.DMA,
            pltpu.SemaphoreType.DMA,
            pltpu.SemaphoreType.DMA,
            pltpu.SemaphoreType.DMA,
        ],
        compiler_params=pltpu.CompilerParams(needs_layout_passes=False),
    )


def _dispatch_body(pos_hbm, gat_hbm, x_hbm, xg_hbm, gsl_hbm,
                   posv, gatv, buf0, buf1, sl0, sl1, ss0, ss1, semg):
    wid = lax.axis_index("s") * 2 + lax.axis_index("c")
    t0 = jnp.bitwise_and(wid, 15) * (PCH * GCH)
    pltpu.sync_copy(pos_hbm.at[wid], posv)
    pltpu.sync_copy(gat_hbm.at[wid], gatv)
    bufs = (buf0, buf1)
    semsL = (sl0, sl1)
    semsS = (ss0, ss1)
    descL = [None, None]
    descS = [None, None]
    gdescs = []
    descL[0] = pltpu.async_copy(x_hbm.at[pl.ds(t0, GCH)], buf0, sl0)
    for c in range(PCH):
        b = c % 2
        nb = (c + 1) % 2
        descL[b].wait()
        if c + 1 < PCH:
            if descS[nb] is not None:
                descS[nb].wait()
            descL[nb] = pltpu.async_copy(
                x_hbm.at[pl.ds(t0 + (c + 1) * GCH, GCH)], bufs[nb], semsL[nb])
        descS[b] = pltpu.async_copy(bufs[b], xg_hbm.at[posv.at[c]], semsS[b])
        gdescs.append(pltpu.async_copy(gatv.at[c], gsl_hbm.at[posv.at[c]],
                                       semg))
    for d in descS:
        d.wait()
    for d in gdescs:
        d.wait()


def _ffn_body(m_ref, xg_ref, w1_ref, b1_ref, w2_ref, b2_ref, g_ref, y_ref):
    i = pl.program_id(0)

    @pl.when(m_ref[1, i] == 1)
    def _():
        xb = xg_ref[...]
        h = lax.dot_general(xb, w1_ref[0], (((1,), (1,)), ((), ())),
                            preferred_element_type=jnp.float32)
        h = jnp.maximum(h + b1_ref[0], 0.0)
        y = lax.dot_general(h, w2_ref[0], (((1,), (1,)), ((), ())),
                            preferred_element_type=jnp.float32)
        y = y + b2_ref[0]
        g = g_ref[0, 0, :]
        y_ref[...] = y * g[:, None]


def _ffn_call(meta, xg, W1, b1r, W2, b2r, g3d):
    grid_spec = pltpu.PrefetchScalarGridSpec(
        num_scalar_prefetch=1,
        grid=(NBLK,),
        in_specs=[
            pl.BlockSpec((BLK, H), lambda i, m: (i, 0)),
            pl.BlockSpec((1, F, H), lambda i, m: (m[0, i], 0, 0)),
            pl.BlockSpec((1, 1, F), lambda i, m: (m[0, i], 0, 0)),
            pl.BlockSpec((1, H, F), lambda i, m: (m[0, i], 0, 0)),
            pl.BlockSpec((1, 1, H), lambda i, m: (m[0, i], 0, 0)),
            pl.BlockSpec((1, 1, BLK), lambda i, m: (i, 0, 0)),
        ],
        out_specs=pl.BlockSpec((BLK, H), lambda i, m: (i, 0)),
    )
    return pl.pallas_call(
        _ffn_body,
        grid_spec=grid_spec,
        out_shape=jax.ShapeDtypeStruct((CAP, H), jnp.float32),
        compiler_params=pltpu.CompilerParams(
            dimension_semantics=("arbitrary",)),
    )(meta, xg, W1, b1r, W2, b2r, g3d)


@functools.cache
def _combine_kernel():
    return pl.kernel(
        _combine_body,
        mesh=_sc_mesh(),
        out_type=jax.ShapeDtypeStruct((N, H), jnp.float32),
        scratch_types=[
            pltpu.VMEM((TOK_W,), jnp.int32),
            pltpu.VMEM((TOK_W,), jnp.int32),
            pltpu.VMEM((CCH, H), jnp.float32),
            pltpu.VMEM((CCH, H), jnp.float32),
            pltpu.VMEM((CCH, H), jnp.float32),
            pltpu.VMEM((CCH, H), jnp.float32),
            pltpu.SemaphoreType.DMA,
            pltpu.SemaphoreType.DMA,
            pltpu.SemaphoreType.DMA,
            pltpu.SemaphoreType.DMA,
            pltpu.SemaphoreType.DMA,
            pltpu.SemaphoreType.DMA,
        ],
        compiler_params=pltpu.CompilerParams(needs_layout_passes=False),
    )


def _combine_body(p0_hbm, p1_hbm, y_hbm, out_hbm, p0v, p1v,
                  a0, b0, a1, b1, sa0, sb0, sa1, sb1, so0, so1):
    wid = lax.axis_index("s") * 2 + lax.axis_index("c")
    base = wid * TOK_W
    pltpu.sync_copy(p0_hbm.at[pl.ds(base, TOK_W)], p0v)
    pltpu.sync_copy(p1_hbm.at[pl.ds(base, TOK_W)], p1v)
    abufs = (a0, a1)
    bbufs = (b0, b1)
    semsA = (sa0, sa1)
    semsB = (sb0, sb1)
    semsO = (so0, so1)
    NCH = TOK_W // CCH
    descA = [None, None]
    descB = [None, None]
    descO = [None, None]

    def _gathers(c, b):
        descA[b] = pltpu.async_copy(
            y_hbm.at[p0v.at[pl.ds(c * CCH, CCH)]], abufs[b], semsA[b])
        descB[b] = pltpu.async_copy(
            y_hbm.at[p1v.at[pl.ds(c * CCH, CCH)]], bbufs[b], semsB[b])

    _gathers(0, 0)
    for c in range(NCH):
        b = c % 2
        nb = (c + 1) % 2
        if c + 1 < NCH:
            if descO[nb] is not None:
                descO[nb].wait()
            _gathers(c + 1, nb)
        descA[b].wait()
        descB[b].wait()
        bufa = abufs[b]
        bufb = bbufs[b]

        @plsc.parallel_loop(0, CCH * (H // 16), unroll=8)
        def _add(j):
            r = j // (H // 16)
            k = j % (H // 16)
            av = bufa[r, pl.ds(k * 16, 16)]
            bv = bufb[r, pl.ds(k * 16, 16)]
            bufa[r, pl.ds(k * 16, 16)] = av + bv
        descO[b] = pltpu.async_copy(
            bufa, out_hbm.at[pl.ds(base + c * CCH, CCH)], semsO[b])
    for d in descO:
        if d is not None:
            d.wait()


def kernel(x, Wr, br, W1, b1, W2, b2):
    x2d = x.reshape(N, H)
    pos_m, gates_m, meta = _router_call(x2d, Wr, br.reshape(1, E))
    pos_pairs = pos_m.reshape(-1)
    gates_pairs = gates_m.reshape(-1)
    xg, gsl = _dispatch_kernel()(pos_pairs.reshape(NW, PCH, GCH),
                                 gates_pairs.reshape(NW, PCH, GCH), x2d)
    y = _ffn_call(meta, xg, W1, b1.reshape(E, 1, F), W2, b2.reshape(E, 1, H),
                  gsl.reshape(NBLK, 1, BLK))
    out2d = _combine_kernel()(pos_pairs[:N], pos_pairs[N:], y)
    return out2d.reshape(x.shape)
